# baseline (speedup 1.0000x reference)
import functools

import jax
import jax.numpy as jnp
from jax import lax
from jax.experimental import pallas as pl
from jax.experimental.pallas import tpu as pltpu
from jax.experimental.pallas import tpu_sc as plsc

NODE_DIM = 128
EDGE_DIM = 16
HIDDEN = 128
N_NODES = 10000
N_EDGES = 320000
TDIM = HIDDEN + EDGE_DIM
NPAD = 10240
NEP = 327680
C = 32
GC = 128
GATHER_S0 = 80


def _sc_geometry():
    try:
        info = plsc.get_sparse_core_info()
        return int(info.num_cores), int(info.num_subcores)
    except Exception:
        return 2, 16


def _tables_tc(x, wcat_s, wcat_d):
    blk = 1000

    def body(x_ref, ws_ref, wd_ref,
             tsm_ref, tdm_ref, tsr_ref, tdr_ref,
             tsm2_ref, tdm2_ref, tsr2_ref, tdr2_ref):
        xb = x_ref[...]
        ts = jnp.dot(xb, ws_ref[...], preferred_element_type=jnp.float32)
        td = jnp.dot(xb, wd_ref[...], preferred_element_type=jnp.float32)
        tsm = ts[:, :HIDDEN].astype(jnp.bfloat16)
        tdm = td[:, :HIDDEN].astype(jnp.bfloat16)
        tsr = ts[:, HIDDEN:]
        tdr = td[:, HIDDEN:]
        tsm_ref[...] = tsm
        tdm_ref[...] = tdm
        tsr_ref[...] = tsr
        tdr_ref[...] = tdr
        tsm2_ref[...] = tsm
        tdm2_ref[...] = tdm
        tsr2_ref[...] = tsr
        tdr2_ref[...] = tdr

    full = lambda i: (i, 0)
    return pl.pallas_call(
        body,
        grid=(N_NODES // blk,),
        in_specs=[
            pl.BlockSpec((blk, NODE_DIM), full),
            pl.BlockSpec((NODE_DIM, TDIM), lambda i: (0, 0)),
            pl.BlockSpec((NODE_DIM, TDIM), lambda i: (0, 0)),
        ],
        out_specs=[
            pl.BlockSpec((blk, HIDDEN), full),
            pl.BlockSpec((blk, HIDDEN), full),
            pl.BlockSpec((blk, EDGE_DIM), full),
            pl.BlockSpec((blk, EDGE_DIM), full),
            pl.BlockSpec((blk, HIDDEN), full),
            pl.BlockSpec((blk, HIDDEN), full),
            pl.BlockSpec((blk, EDGE_DIM), full),
            pl.BlockSpec((blk, EDGE_DIM), full),
        ],
        out_shape=[
            jax.ShapeDtypeStruct((N_NODES, HIDDEN), jnp.bfloat16),
            jax.ShapeDtypeStruct((N_NODES, HIDDEN), jnp.bfloat16),
            jax.ShapeDtypeStruct((N_NODES, EDGE_DIM), jnp.float32),
            jax.ShapeDtypeStruct((N_NODES, EDGE_DIM), jnp.float32),
            jax.ShapeDtypeStruct((N_NODES, HIDDEN), jnp.bfloat16),
            jax.ShapeDtypeStruct((N_NODES, HIDDEN), jnp.bfloat16),
            jax.ShapeDtypeStruct((N_NODES, EDGE_DIM), jnp.float32),
            jax.ShapeDtypeStruct((N_NODES, EDGE_DIM), jnp.float32),
        ],
    )(x, wcat_s, wcat_d)


def _build_gather(nc, ns, s0_frac=0.5):
    nw = nc * ns
    spw = NEP // (GC * nw) * nc
    if nc == 2:
        s_per = (GATHER_S0, spw - GATHER_S0)
    else:
        s_per = (spw,)
    smax = max(s_per)
    mesh = plsc.VectorSubcoreMesh(core_axis_name="c", subcore_axis_name="s",
                                  num_cores=nc, num_subcores=ns)

    @functools.partial(
        pl.kernel,
        out_type=(
            jax.ShapeDtypeStruct((NEP, HIDDEN), jnp.bfloat16),
            jax.ShapeDtypeStruct((NEP, HIDDEN), jnp.bfloat16),
            jax.ShapeDtypeStruct((NEP // 8, HIDDEN), jnp.float32),
        ),
        mesh=mesh,
        compiler_params=pltpu.CompilerParams(use_tc_tiling_on_sc=False),
        scratch_types=[
            pltpu.VMEM((smax, GC), jnp.int32),
            pltpu.VMEM((smax, GC), jnp.int32),
            pltpu.VMEM((2, GC, HIDDEN), jnp.bfloat16),
            pltpu.VMEM((2, GC, HIDDEN), jnp.bfloat16),
            pltpu.VMEM((2, GC, EDGE_DIM), jnp.float32),
            pltpu.VMEM((2, GC, EDGE_DIM), jnp.float32),
            pltpu.VMEM((2, GC // 8, HIDDEN), jnp.float32),
            pltpu.SemaphoreType.DMA,
            pltpu.SemaphoreType.DMA,
            pltpu.SemaphoreType.DMA,
            pltpu.SemaphoreType.DMA,
        ],
    )
    def gather_k(tsm_hbm, tdm_hbm, tsr_hbm, tdr_hbm,
                 tsm2_hbm, tdm2_hbm, tsr2_hbm, tdr2_hbm,
                 s2_hbm, d2_hbm,
                 gms_hbm, gmd_hbm, grp_hbm,
                 sidx, didx, smb, dmb, srb, drb, rbuf,
                 gsem0, gsem1, wsem0, wsem1):
        cid = lax.axis_index("c")
        sid = lax.axis_index("s")

        if nc == 2:
            sbase = jnp.where(cid == 0, sid * s_per[0],
                              ns * s_per[0] + sid * s_per[1])
            my_s = jnp.where(cid == 0, s_per[0], s_per[1])

            @pl.when(cid == 0)
            def _():
                pltpu.sync_copy(s2_hbm.at[pl.ds(sid * s_per[0], s_per[0])],
                                sidx.at[pl.ds(0, s_per[0])])
                pltpu.sync_copy(d2_hbm.at[pl.ds(sid * s_per[0], s_per[0])],
                                didx.at[pl.ds(0, s_per[0])])

            @pl.when(cid == 1)
            def _():
                b = ns * s_per[0] + sid * s_per[1]
                pltpu.sync_copy(s2_hbm.at[pl.ds(b, s_per[1])],
                                sidx.at[pl.ds(0, s_per[1])])
                pltpu.sync_copy(d2_hbm.at[pl.ds(b, s_per[1])],
                                didx.at[pl.ds(0, s_per[1])])
        else:
            sbase = sid * s_per[0]
            my_s = s_per[0]
            pltpu.sync_copy(s2_hbm.at[pl.ds(sbase, s_per[0])], sidx)
            pltpu.sync_copy(d2_hbm.at[pl.ds(sbase, s_per[0])], didx)

        gsems = (gsem0, gsem1)
        wsems = (wsem0, wsem1)

        def fire(s, p):
            @pl.when(cid == 0)
            def _():
                pltpu.async_copy(tsm_hbm.at[sidx.at[s]], smb.at[p], gsems[p])
                pltpu.async_copy(tdm_hbm.at[didx.at[s]], dmb.at[p], gsems[p])
                pltpu.async_copy(tsr_hbm.at[sidx.at[s]], srb.at[p], gsems[p])
                pltpu.async_copy(tdr_hbm.at[didx.at[s]], drb.at[p], gsems[p])

            @pl.when(cid == 1)
            def _():
                pltpu.async_copy(tsm2_hbm.at[sidx.at[s]], smb.at[p], gsems[p])
                pltpu.async_copy(tdm2_hbm.at[didx.at[s]], dmb.at[p], gsems[p])
                pltpu.async_copy(tsr2_hbm.at[sidx.at[s]], srb.at[p], gsems[p])
                pltpu.async_copy(tdr2_hbm.at[didx.at[s]], drb.at[p], gsems[p])

        def drain_gather(s, p):
            pltpu.make_async_copy(tsm_hbm.at[sidx.at[s]], smb.at[p],
                                  gsems[p]).wait()
            pltpu.make_async_copy(tdm_hbm.at[didx.at[s]], dmb.at[p],
                                  gsems[p]).wait()
            pltpu.make_async_copy(tsr_hbm.at[sidx.at[s]], srb.at[p],
                                  gsems[p]).wait()
            pltpu.make_async_copy(tdr_hbm.at[didx.at[s]], drb.at[p],
                                  gsems[p]).wait()

        def res_sum(p):
            def rbody(r2, carry):
                for q in range(8):
                    rbuf[p, r2, pl.ds(q * EDGE_DIM, EDGE_DIM)] = (
                        srb[p, r2 * 8 + q, :] + drb[p, r2 * 8 + q, :])
                return carry

            lax.fori_loop(0, GC // 8, rbody, 0)

        def _wcopies(g, p):
            r = pl.ds(g * GC, GC)
            rp = pl.ds(g * (GC // 8), GC // 8)
            return (
                (smb.at[p], gms_hbm.at[r], wsems[p]),
                (dmb.at[p], gmd_hbm.at[r], wsems[p]),
                (rbuf.at[p], grp_hbm.at[rp], wsems[p]),
            )

        def write(g, p):
            for a, b, sem in _wcopies(g, p):
                pltpu.async_copy(a, b, sem)

        def drain_write(g, p):
            for a, b, sem in _wcopies(g, p):
                pltpu.make_async_copy(a, b, sem).wait()

        fire(0, 0)

        @pl.when(my_s > 1)
        def _():
            fire(1, 1)

        def body(t, carry):
            for p in range(2):
                s = 2 * t + p

                @pl.when(s < my_s)
                def _():
                    drain_gather(s, p)
                    res_sum(p)
                    write(sbase + s, p)
                    drain_write(sbase + s, p)
                    nxt = s + 2

                    @pl.when(nxt < my_s)
                    def _():
                        fire(nxt, p)

            return carry

        lax.fori_loop(0, (smax + 1) // 2, body, 0)

    return gather_k


def _unpack16(gp, blk):
    parts = jnp.stack(
        [gp[:, q * EDGE_DIM:(q + 1) * EDGE_DIM] for q in range(8)], axis=1)
    return jnp.reshape(parts, (blk, EDGE_DIM))


def _pack16(ne, blk):
    r = jnp.reshape(ne, (blk // 8, 8, EDGE_DIM))
    return jnp.concatenate([r[:, q, :] for q in range(8)], axis=1)


def _edge_tc(gms, gmd, grp, ea, w1e, b1, w2, wre, bc, gamma, beta):
    blk = 2048

    def body(gms_ref, gmd_ref, grp_ref, ea_ref, w1e_ref, b1_ref,
             w2_ref, wre_ref, bc_ref, g_ref, be_ref, o_ref, op_ref):
        eab = ea_ref[...].astype(jnp.bfloat16)
        h = (gms_ref[...].astype(jnp.float32) + gmd_ref[...].astype(jnp.float32)
             + jnp.dot(eab, w1e_ref[...], preferred_element_type=jnp.float32)
             + b1_ref[...])
        h = h * (1.0 / (1.0 + jnp.exp(-h)))
        o = (jnp.dot(h.astype(jnp.bfloat16), w2_ref[...],
                     preferred_element_type=jnp.float32)
             + jnp.dot(eab, wre_ref[...], preferred_element_type=jnp.float32)
             + _unpack16(grp_ref[...], blk) + bc_ref[...])
        m = jnp.mean(o, axis=1, keepdims=True)
        cde = o - m
        v = jnp.mean(cde * cde, axis=1, keepdims=True)
        ne = cde * lax.rsqrt(v + 1e-5) * g_ref[...] + be_ref[...]
        o_ref[...] = ne
        op_ref[...] = _pack16(ne, blk)

    return pl.pallas_call(
        body,
        grid=(-(-N_EDGES // blk),),
        in_specs=[
            pl.BlockSpec((blk, HIDDEN), lambda i: (i, 0)),
            pl.BlockSpec((blk, HIDDEN), lambda i: (i, 0)),
            pl.BlockSpec((blk // 8, HIDDEN), lambda i: (i, 0)),
            pl.BlockSpec((blk, EDGE_DIM), lambda i: (i, 0)),
            pl.BlockSpec((EDGE_DIM, HIDDEN), lambda i: (0, 0)),
            pl.BlockSpec((1, HIDDEN), lambda i: (0, 0)),
            pl.BlockSpec((HIDDEN, EDGE_DIM), lambda i: (0, 0)),
            pl.BlockSpec((EDGE_DIM, EDGE_DIM), lambda i: (0, 0)),
            pl.BlockSpec((1, EDGE_DIM), lambda i: (0, 0)),
            pl.BlockSpec((1, EDGE_DIM), lambda i: (0, 0)),
            pl.BlockSpec((1, EDGE_DIM), lambda i: (0, 0)),
        ],
        out_specs=[
            pl.BlockSpec((blk, EDGE_DIM), lambda i: (i, 0)),
            pl.BlockSpec((blk // 8, HIDDEN), lambda i: (i, 0)),
        ],
        out_shape=[
            jax.ShapeDtypeStruct((N_EDGES, EDGE_DIM), jnp.float32),
            jax.ShapeDtypeStruct((NEP // 8, HIDDEN), jnp.float32),
        ],
    )(gms, gmd, grp, ea, w1e, b1, w2, wre, bc, gamma, beta)


def _build_scatter(nc, ns):
    nw = nc * ns
    pw = NEP // nw
    c2 = 64
    ki = 40
    chunk = ki * c2
    outer = pw // chunk
    rows_per = NPAD // ns
    mesh = plsc.VectorSubcoreMesh(core_axis_name="c", subcore_axis_name="s",
                                  num_cores=nc, num_subcores=ns)

    @functools.partial(
        pl.kernel,
        out_type=jax.ShapeDtypeStruct((nc, NPAD, EDGE_DIM), jnp.float32),
        mesh=mesh,
        compiler_params=pltpu.CompilerParams(use_tc_tiling_on_sc=False),
        scratch_types=[
            pltpu.VMEM((ki, c2), jnp.int32),
            pltpu.VMEM((chunk // 8, HIDDEN), jnp.float32),
            pltpu.VMEM((chunk, EDGE_DIM), jnp.float32),
            pltpu.VMEM((rows_per, EDGE_DIM), jnp.float32),
            pltpu.VMEM_SHARED((NPAD, EDGE_DIM), jnp.float32),
        ],
    )
    def scatter_k(nep_hbm, d2_hbm, out_hbm, idxb, rowsb128, rowsb, bounce,
                  aggsh):
        cid = lax.axis_index("c")
        sid = lax.axis_index("s")
        z = jnp.zeros((16,), jnp.float32)

        def zbody(i, carry):
            bounce[i, :] = z
            return carry

        lax.fori_loop(0, rows_per, zbody, 0)
        pltpu.sync_copy(bounce, aggsh.at[pl.ds(sid * rows_per, rows_per)])
        plsc.subcore_barrier()

        wid = cid * ns + sid
        e0 = wid * pw
        row0 = e0 // c2
        prow0 = e0 // 8
        pchunk = chunk // 8

        def body(t, carry):
            pltpu.sync_copy(d2_hbm.at[pl.ds(row0 + t * ki, ki)], idxb)
            pltpu.sync_copy(nep_hbm.at[pl.ds(prow0 + t * pchunk, pchunk)],
                            rowsb128)

            def rbody(r, carry2):
                for q in range(8):
                    rowsb[r * 8 + q, :] = rowsb128[r, pl.ds(q * EDGE_DIM,
                                                            EDGE_DIM)]
                return carry2

            lax.fori_loop(0, pchunk, rbody, 0)
            for j in range(ki):
                pltpu.sync_copy(rowsb.at[pl.ds(j * c2, c2)],
                                aggsh.at[idxb.at[j]], add=True)
            return carry

        lax.fori_loop(0, outer, body, 0)
        plsc.subcore_barrier()
        pltpu.sync_copy(aggsh.at[pl.ds(sid * rows_per, rows_per)], bounce)
        pltpu.sync_copy(bounce, out_hbm.at[cid, pl.ds(sid * rows_per, rows_per)])

    return scatter_k


def _node_tc(x, aggp, w1x, w1a, b1, w2, wrx, wra, bc, gamma, beta):
    blk = 1000
    nc = aggp.shape[0]

    def body(x_ref, ap_ref, w1x_ref, w1a_ref, b1_ref, w2_ref, wrx_ref,
             wra_ref, bc_ref, g_ref, be_ref, o_ref):
        xb = x_ref[...]
        a = jnp.sum(ap_ref[...], axis=0)
        h = (jnp.dot(xb, w1x_ref[...], preferred_element_type=jnp.float32)
             + jnp.dot(a, w1a_ref[...], preferred_element_type=jnp.float32)
             + b1_ref[...])
        h = h * (1.0 / (1.0 + jnp.exp(-h)))
        o = (jnp.dot(h, w2_ref[...], preferred_element_type=jnp.float32)
             + jnp.dot(xb, wrx_ref[...], preferred_element_type=jnp.float32)
             + jnp.dot(a, wra_ref[...], preferred_element_type=jnp.float32)
             + bc_ref[...])
        m = jnp.mean(o, axis=1, keepdims=True)
        cde = o - m
        v = jnp.mean(cde * cde, axis=1, keepdims=True)
        o_ref[...] = cde * lax.rsqrt(v + 1e-5) * g_ref[...] + be_ref[...]

    return pl.pallas_call(
        body,
        grid=(N_NODES // blk,),
        in_specs=[
            pl.BlockSpec((blk, NODE_DIM), lambda i: (i, 0)),
            pl.BlockSpec((nc, blk, EDGE_DIM), lambda i: (0, i, 0)),
            pl.BlockSpec((NODE_DIM, HIDDEN), lambda i: (0, 0)),
            pl.BlockSpec((EDGE_DIM, HIDDEN), lambda i: (0, 0)),
            pl.BlockSpec((1, HIDDEN), lambda i: (0, 0)),
            pl.BlockSpec((HIDDEN, NODE_DIM), lambda i: (0, 0)),
            pl.BlockSpec((NODE_DIM, NODE_DIM), lambda i: (0, 0)),
            pl.BlockSpec((EDGE_DIM, NODE_DIM), lambda i: (0, 0)),
            pl.BlockSpec((1, NODE_DIM), lambda i: (0, 0)),
            pl.BlockSpec((1, NODE_DIM), lambda i: (0, 0)),
            pl.BlockSpec((1, NODE_DIM), lambda i: (0, 0)),
        ],
        out_specs=pl.BlockSpec((blk, NODE_DIM), lambda i: (i, 0)),
        out_shape=jax.ShapeDtypeStruct((N_NODES, NODE_DIM), jnp.float32),
    )(x, aggp, w1x, w1a, b1, w2, wrx, wra, bc, gamma, beta)


def kernel(x, edge_attr, edge_index,
           edge_W1, edge_b1, edge_W2, edge_b2, edge_Wres, edge_bres,
           edge_gamma, edge_beta,
           node_W1, node_b1, node_W2, node_b2, node_Wres, node_bres,
           node_gamma, node_beta):
    nc, ns = _sc_geometry()

    npad_e = NEP - N_EDGES
    src_p = jnp.concatenate(
        [edge_index[0], jnp.zeros((npad_e,), jnp.int32)])
    dst_p = jnp.concatenate(
        [edge_index[1], jnp.full((npad_e,), N_NODES, jnp.int32)])
    src2g = src_p.reshape(NEP // GC, GC)
    dst2g = dst_p.reshape(NEP // GC, GC)
    dst2 = dst_p.reshape(NEP // 64, 64)
    ea_p = jnp.concatenate(
        [edge_attr, jnp.zeros((npad_e, EDGE_DIM), jnp.float32)])

    w1e = edge_W1[:EDGE_DIM]
    wcat_s = jnp.concatenate(
        [edge_W1[EDGE_DIM:EDGE_DIM + NODE_DIM],
         edge_Wres[EDGE_DIM:EDGE_DIM + NODE_DIM]], axis=1)
    wcat_d = jnp.concatenate(
        [edge_W1[EDGE_DIM + NODE_DIM:],
         edge_Wres[EDGE_DIM + NODE_DIM:]], axis=1)
    wre = edge_Wres[:EDGE_DIM]
    ebc = (edge_b2 + edge_bres).reshape(1, EDGE_DIM)

    tabs = _tables_tc(x, wcat_s, wcat_d)
    gms, gmd, grp = _build_gather(nc, ns)(tabs[0], tabs[1], tabs[2], tabs[3],
                                          tabs[4], tabs[5], tabs[6], tabs[7],
                                          src2g, dst2g)
    new_edge, nep = _edge_tc(gms, gmd, grp, ea_p,
                             w1e.astype(jnp.bfloat16),
                             edge_b1.reshape(1, HIDDEN),
                             edge_W2.astype(jnp.bfloat16),
                             wre.astype(jnp.bfloat16), ebc,
                             edge_gamma.reshape(1, EDGE_DIM),
                             edge_beta.reshape(1, EDGE_DIM))
    aggp = _build_scatter(nc, ns)(nep, dst2)

    nbc = (node_b2 + node_bres).reshape(1, NODE_DIM)
    new_x = _node_tc(
        x, aggp[:, :N_NODES, :],
        node_W1[:NODE_DIM], node_W1[NODE_DIM:],
        node_b1.reshape(1, HIDDEN),
        node_W2, node_Wres[:NODE_DIM], node_Wres[NODE_DIM:], nbc,
        node_gamma.reshape(1, NODE_DIM), node_beta.reshape(1, NODE_DIM))
    return (new_x, new_edge)


# speedup vs baseline: 1.2548x; 1.2548x over previous
import functools

import jax
import jax.numpy as jnp
from jax import lax
from jax.experimental import pallas as pl
from jax.experimental.pallas import tpu as pltpu
from jax.experimental.pallas import tpu_sc as plsc

NODE_DIM = 128
EDGE_DIM = 16
HIDDEN = 128
N_NODES = 10000
N_EDGES = 320000
TDIM = HIDDEN + EDGE_DIM
NPAD = 10240
NEP = 327680
C = 32
GC = 128
GATHER_S0 = 80


def _sc_geometry():
    try:
        info = plsc.get_sparse_core_info()
        return int(info.num_cores), int(info.num_subcores)
    except Exception:
        return 2, 16


def _tables_tc(x, wcat_s, wcat_d):
    blk = 1000

    def body(x_ref, ws_ref, wd_ref, ts_ref, td_ref, ts2_ref, td2_ref):
        xb = x_ref[...]
        ts = jnp.dot(xb, ws_ref[...], preferred_element_type=jnp.float32)
        td = jnp.dot(xb, wd_ref[...], preferred_element_type=jnp.float32)
        ts_ref[...] = ts
        td_ref[...] = td
        ts2_ref[...] = ts
        td2_ref[...] = td

    full = lambda i: (i, 0)
    return pl.pallas_call(
        body,
        grid=(N_NODES // blk,),
        in_specs=[
            pl.BlockSpec((blk, NODE_DIM), full),
            pl.BlockSpec((NODE_DIM, TDIM), lambda i: (0, 0)),
            pl.BlockSpec((NODE_DIM, TDIM), lambda i: (0, 0)),
        ],
        out_specs=[
            pl.BlockSpec((blk, TDIM), full),
            pl.BlockSpec((blk, TDIM), full),
            pl.BlockSpec((blk, TDIM), full),
            pl.BlockSpec((blk, TDIM), full),
        ],
        out_shape=[
            jax.ShapeDtypeStruct((N_NODES, TDIM), jnp.float32),
            jax.ShapeDtypeStruct((N_NODES, TDIM), jnp.float32),
            jax.ShapeDtypeStruct((N_NODES, TDIM), jnp.float32),
            jax.ShapeDtypeStruct((N_NODES, TDIM), jnp.float32),
        ],
    )(x, wcat_s, wcat_d)


def _build_gather(nc, ns, s0_frac=0.5):
    nw = nc * ns
    spw = NEP // (GC * nw) * nc
    if nc == 2:
        s_per = (GATHER_S0, spw - GATHER_S0)
    else:
        s_per = (spw,)
    smax = max(s_per)
    mesh = plsc.VectorSubcoreMesh(core_axis_name="c", subcore_axis_name="s",
                                  num_cores=nc, num_subcores=ns)

    @functools.partial(
        pl.kernel,
        out_type=(
            jax.ShapeDtypeStruct((NEP, HIDDEN), jnp.float32),
            jax.ShapeDtypeStruct((NEP, HIDDEN), jnp.float32),
            jax.ShapeDtypeStruct((NEP // 8, HIDDEN), jnp.float32),
        ),
        mesh=mesh,
        compiler_params=pltpu.CompilerParams(use_tc_tiling_on_sc=False),
        scratch_types=[
            pltpu.VMEM((smax, GC), jnp.int32),
            pltpu.VMEM((smax, GC), jnp.int32),
            pltpu.VMEM((2, GC, TDIM), jnp.float32),
            pltpu.VMEM((2, GC, TDIM), jnp.float32),
            pltpu.VMEM((2, GC // 8, HIDDEN), jnp.float32),
            pltpu.SemaphoreType.DMA,
            pltpu.SemaphoreType.DMA,
            pltpu.SemaphoreType.DMA,
            pltpu.SemaphoreType.DMA,
        ],
    )
    def gather_k(ts_hbm, td_hbm, ts2_hbm, td2_hbm, s2_hbm, d2_hbm,
                 gms_hbm, gmd_hbm, grp_hbm,
                 sidx, didx, sbuf, dbuf, rbuf, gsem0, gsem1, wsem0, wsem1):
        cid = lax.axis_index("c")
        sid = lax.axis_index("s")

        if nc == 2:
            sbase = jnp.where(cid == 0, sid * s_per[0],
                              ns * s_per[0] + sid * s_per[1])
            my_s = jnp.where(cid == 0, s_per[0], s_per[1])

            @pl.when(cid == 0)
            def _():
                pltpu.sync_copy(s2_hbm.at[pl.ds(sid * s_per[0], s_per[0])],
                                sidx.at[pl.ds(0, s_per[0])])
                pltpu.sync_copy(d2_hbm.at[pl.ds(sid * s_per[0], s_per[0])],
                                didx.at[pl.ds(0, s_per[0])])

            @pl.when(cid == 1)
            def _():
                b = ns * s_per[0] + sid * s_per[1]
                pltpu.sync_copy(s2_hbm.at[pl.ds(b, s_per[1])],
                                sidx.at[pl.ds(0, s_per[1])])
                pltpu.sync_copy(d2_hbm.at[pl.ds(b, s_per[1])],
                                didx.at[pl.ds(0, s_per[1])])
        else:
            sbase = sid * s_per[0]
            my_s = s_per[0]
            pltpu.sync_copy(s2_hbm.at[pl.ds(sbase, s_per[0])], sidx)
            pltpu.sync_copy(d2_hbm.at[pl.ds(sbase, s_per[0])], didx)

        gsems = (gsem0, gsem1)
        wsems = (wsem0, wsem1)

        def fire(s, p):
            @pl.when(cid == 0)
            def _():
                pltpu.async_copy(ts_hbm.at[sidx.at[s]], sbuf.at[p], gsems[p])
                pltpu.async_copy(td_hbm.at[didx.at[s]], dbuf.at[p], gsems[p])

            @pl.when(cid == 1)
            def _():
                pltpu.async_copy(ts2_hbm.at[sidx.at[s]], sbuf.at[p], gsems[p])
                pltpu.async_copy(td2_hbm.at[didx.at[s]], dbuf.at[p], gsems[p])

        def drain_gather(s, p):
            pltpu.make_async_copy(ts_hbm.at[sidx.at[s]], sbuf.at[p],
                                  gsems[p]).wait()
            pltpu.make_async_copy(td_hbm.at[didx.at[s]], dbuf.at[p],
                                  gsems[p]).wait()

        def res_sum(p):
            def rbody(r2, carry):
                for q in range(8):
                    rbuf[p, r2, pl.ds(q * EDGE_DIM, EDGE_DIM)] = (
                        sbuf[p, r2 * 8 + q, pl.ds(HIDDEN, EDGE_DIM)]
                        + dbuf[p, r2 * 8 + q, pl.ds(HIDDEN, EDGE_DIM)])
                return carry

            lax.fori_loop(0, GC // 8, rbody, 0)

        def _wcopies(g, p):
            r = pl.ds(g * GC, GC)
            rp = pl.ds(g * (GC // 8), GC // 8)
            return (
                (sbuf.at[p, :, pl.ds(0, HIDDEN)], gms_hbm.at[r], wsems[p]),
                (dbuf.at[p, :, pl.ds(0, HIDDEN)], gmd_hbm.at[r], wsems[p]),
                (rbuf.at[p], grp_hbm.at[rp], wsems[p]),
            )

        def write(g, p):
            for a, b, sem in _wcopies(g, p):
                pltpu.async_copy(a, b, sem)

        def drain_write(g, p):
            for a, b, sem in _wcopies(g, p):
                pltpu.make_async_copy(a, b, sem).wait()

        fire(0, 0)

        @pl.when(my_s > 1)
        def _():
            fire(1, 1)

        def body(t, carry):
            for p in range(2):
                s = 2 * t + p

                @pl.when(s < my_s)
                def _():
                    drain_gather(s, p)
                    res_sum(p)
                    write(sbase + s, p)
                    drain_write(sbase + s, p)
                    nxt = s + 2

                    @pl.when(nxt < my_s)
                    def _():
                        fire(nxt, p)

            return carry

        lax.fori_loop(0, (smax + 1) // 2, body, 0)

    return gather_k


def _unpack16(gp, blk):
    parts = jnp.stack(
        [gp[:, q * EDGE_DIM:(q + 1) * EDGE_DIM] for q in range(8)], axis=1)
    return jnp.reshape(parts, (blk, EDGE_DIM))


def _pack16(ne, blk):
    r = jnp.reshape(ne, (blk // 8, 8, EDGE_DIM))
    return jnp.concatenate([r[:, q, :] for q in range(8)], axis=1)


def _edge_tc(gms, gmd, grp, ea, w1e, b1, w2, wre, bc, gamma, beta):
    blk = 2048

    def body(gms_ref, gmd_ref, grp_ref, ea_ref, w1e_ref, b1_ref,
             w2_ref, wre_ref, bc_ref, g_ref, be_ref, o_ref, op_ref):
        eab = ea_ref[...].astype(jnp.bfloat16)
        h = (gms_ref[...] + gmd_ref[...]
             + jnp.dot(eab, w1e_ref[...], preferred_element_type=jnp.float32)
             + b1_ref[...])
        h = h * (1.0 / (1.0 + jnp.exp(-h)))
        o = (jnp.dot(h.astype(jnp.bfloat16), w2_ref[...],
                     preferred_element_type=jnp.float32)
             + jnp.dot(eab, wre_ref[...], preferred_element_type=jnp.float32)
             + _unpack16(grp_ref[...], blk) + bc_ref[...])
        m = jnp.mean(o, axis=1, keepdims=True)
        cde = o - m
        v = jnp.mean(cde * cde, axis=1, keepdims=True)
        ne = cde * lax.rsqrt(v + 1e-5) * g_ref[...] + be_ref[...]
        o_ref[...] = ne
        op_ref[...] = _pack16(ne, blk)

    return pl.pallas_call(
        body,
        grid=(-(-N_EDGES // blk),),
        in_specs=[
            pl.BlockSpec((blk, HIDDEN), lambda i: (i, 0)),
            pl.BlockSpec((blk, HIDDEN), lambda i: (i, 0)),
            pl.BlockSpec((blk // 8, HIDDEN), lambda i: (i, 0)),
            pl.BlockSpec((blk, EDGE_DIM), lambda i: (i, 0)),
            pl.BlockSpec((EDGE_DIM, HIDDEN), lambda i: (0, 0)),
            pl.BlockSpec((1, HIDDEN), lambda i: (0, 0)),
            pl.BlockSpec((HIDDEN, EDGE_DIM), lambda i: (0, 0)),
            pl.BlockSpec((EDGE_DIM, EDGE_DIM), lambda i: (0, 0)),
            pl.BlockSpec((1, EDGE_DIM), lambda i: (0, 0)),
            pl.BlockSpec((1, EDGE_DIM), lambda i: (0, 0)),
            pl.BlockSpec((1, EDGE_DIM), lambda i: (0, 0)),
        ],
        out_specs=[
            pl.BlockSpec((blk, EDGE_DIM), lambda i: (i, 0)),
            pl.BlockSpec((blk // 8, HIDDEN), lambda i: (i, 0)),
        ],
        out_shape=[
            jax.ShapeDtypeStruct((N_EDGES, EDGE_DIM), jnp.float32),
            jax.ShapeDtypeStruct((NEP // 8, HIDDEN), jnp.float32),
        ],
    )(gms, gmd, grp, ea, w1e, b1, w2, wre, bc, gamma, beta)


def _build_scatter(nc, ns):
    nw = nc * ns
    pw = NEP // nw
    c2 = 64
    ki = 40
    chunk = ki * c2
    outer = pw // chunk
    rows_per = NPAD // ns
    mesh = plsc.VectorSubcoreMesh(core_axis_name="c", subcore_axis_name="s",
                                  num_cores=nc, num_subcores=ns)

    @functools.partial(
        pl.kernel,
        out_type=jax.ShapeDtypeStruct((nc, NPAD, EDGE_DIM), jnp.float32),
        mesh=mesh,
        compiler_params=pltpu.CompilerParams(use_tc_tiling_on_sc=False),
        scratch_types=[
            pltpu.VMEM((ki, c2), jnp.int32),
            pltpu.VMEM((chunk // 8, HIDDEN), jnp.float32),
            pltpu.VMEM((chunk, EDGE_DIM), jnp.float32),
            pltpu.VMEM((rows_per, EDGE_DIM), jnp.float32),
            pltpu.VMEM_SHARED((NPAD, EDGE_DIM), jnp.float32),
        ],
    )
    def scatter_k(nep_hbm, d2_hbm, out_hbm, idxb, rowsb128, rowsb, bounce,
                  aggsh):
        cid = lax.axis_index("c")
        sid = lax.axis_index("s")
        z = jnp.zeros((16,), jnp.float32)

        def zbody(i, carry):
            bounce[i, :] = z
            return carry

        lax.fori_loop(0, rows_per, zbody, 0)
        pltpu.sync_copy(bounce, aggsh.at[pl.ds(sid * rows_per, rows_per)])
        plsc.subcore_barrier()

        wid = cid * ns + sid
        e0 = wid * pw
        row0 = e0 // c2
        prow0 = e0 // 8
        pchunk = chunk // 8

        def body(t, carry):
            pltpu.sync_copy(d2_hbm.at[pl.ds(row0 + t * ki, ki)], idxb)
            pltpu.sync_copy(nep_hbm.at[pl.ds(prow0 + t * pchunk, pchunk)],
                            rowsb128)

            def rbody(r, carry2):
                for q in range(8):
                    rowsb[r * 8 + q, :] = rowsb128[r, pl.ds(q * EDGE_DIM,
                                                            EDGE_DIM)]
                return carry2

            lax.fori_loop(0, pchunk, rbody, 0)
            for j in range(ki):
                pltpu.sync_copy(rowsb.at[pl.ds(j * c2, c2)],
                                aggsh.at[idxb.at[j]], add=True)
            return carry

        lax.fori_loop(0, outer, body, 0)
        plsc.subcore_barrier()
        pltpu.sync_copy(aggsh.at[pl.ds(sid * rows_per, rows_per)], bounce)
        pltpu.sync_copy(bounce, out_hbm.at[cid, pl.ds(sid * rows_per, rows_per)])

    return scatter_k


def _node_tc(x, aggp, w1x, w1a, b1, w2, wrx, wra, bc, gamma, beta):
    blk = 1000
    nc = aggp.shape[0]

    def body(x_ref, ap_ref, w1x_ref, w1a_ref, b1_ref, w2_ref, wrx_ref,
             wra_ref, bc_ref, g_ref, be_ref, o_ref):
        xb = x_ref[...]
        a = jnp.sum(ap_ref[...], axis=0)
        h = (jnp.dot(xb, w1x_ref[...], preferred_element_type=jnp.float32)
             + jnp.dot(a, w1a_ref[...], preferred_element_type=jnp.float32)
             + b1_ref[...])
        h = h * (1.0 / (1.0 + jnp.exp(-h)))
        o = (jnp.dot(h, w2_ref[...], preferred_element_type=jnp.float32)
             + jnp.dot(xb, wrx_ref[...], preferred_element_type=jnp.float32)
             + jnp.dot(a, wra_ref[...], preferred_element_type=jnp.float32)
             + bc_ref[...])
        m = jnp.mean(o, axis=1, keepdims=True)
        cde = o - m
        v = jnp.mean(cde * cde, axis=1, keepdims=True)
        o_ref[...] = cde * lax.rsqrt(v + 1e-5) * g_ref[...] + be_ref[...]

    return pl.pallas_call(
        body,
        grid=(N_NODES // blk,),
        in_specs=[
            pl.BlockSpec((blk, NODE_DIM), lambda i: (i, 0)),
            pl.BlockSpec((nc, blk, EDGE_DIM), lambda i: (0, i, 0)),
            pl.BlockSpec((NODE_DIM, HIDDEN), lambda i: (0, 0)),
            pl.BlockSpec((EDGE_DIM, HIDDEN), lambda i: (0, 0)),
            pl.BlockSpec((1, HIDDEN), lambda i: (0, 0)),
            pl.BlockSpec((HIDDEN, NODE_DIM), lambda i: (0, 0)),
            pl.BlockSpec((NODE_DIM, NODE_DIM), lambda i: (0, 0)),
            pl.BlockSpec((EDGE_DIM, NODE_DIM), lambda i: (0, 0)),
            pl.BlockSpec((1, NODE_DIM), lambda i: (0, 0)),
            pl.BlockSpec((1, NODE_DIM), lambda i: (0, 0)),
            pl.BlockSpec((1, NODE_DIM), lambda i: (0, 0)),
        ],
        out_specs=pl.BlockSpec((blk, NODE_DIM), lambda i: (i, 0)),
        out_shape=jax.ShapeDtypeStruct((N_NODES, NODE_DIM), jnp.float32),
    )(x, aggp, w1x, w1a, b1, w2, wrx, wra, bc, gamma, beta)


def kernel(x, edge_attr, edge_index,
           edge_W1, edge_b1, edge_W2, edge_b2, edge_Wres, edge_bres,
           edge_gamma, edge_beta,
           node_W1, node_b1, node_W2, node_b2, node_Wres, node_bres,
           node_gamma, node_beta):
    nc, ns = _sc_geometry()

    npad_e = NEP - N_EDGES
    src_p = jnp.concatenate(
        [edge_index[0], jnp.zeros((npad_e,), jnp.int32)])
    dst_p = jnp.concatenate(
        [edge_index[1], jnp.full((npad_e,), N_NODES, jnp.int32)])
    src2g = src_p.reshape(NEP // GC, GC)
    dst2g = dst_p.reshape(NEP // GC, GC)
    dst2 = dst_p.reshape(NEP // 64, 64)
    ea_p = jnp.concatenate(
        [edge_attr, jnp.zeros((npad_e, EDGE_DIM), jnp.float32)])

    w1e = edge_W1[:EDGE_DIM]
    wcat_s = jnp.concatenate(
        [edge_W1[EDGE_DIM:EDGE_DIM + NODE_DIM],
         edge_Wres[EDGE_DIM:EDGE_DIM + NODE_DIM]], axis=1)
    wcat_d = jnp.concatenate(
        [edge_W1[EDGE_DIM + NODE_DIM:],
         edge_Wres[EDGE_DIM + NODE_DIM:]], axis=1)
    wre = edge_Wres[:EDGE_DIM]
    ebc = (edge_b2 + edge_bres).reshape(1, EDGE_DIM)

    tsrc, tdst, tsrc2, tdst2 = _tables_tc(x, wcat_s, wcat_d)
    gms, gmd, grp = _build_gather(nc, ns)(tsrc, tdst, tsrc2, tdst2,
                                          src2g, dst2g)
    new_edge, nep = _edge_tc(gms, gmd, grp, ea_p,
                             w1e.astype(jnp.bfloat16),
                             edge_b1.reshape(1, HIDDEN),
                             edge_W2.astype(jnp.bfloat16),
                             wre.astype(jnp.bfloat16), ebc,
                             edge_gamma.reshape(1, EDGE_DIM),
                             edge_beta.reshape(1, EDGE_DIM))
    aggp = _build_scatter(nc, ns)(nep, dst2)

    nbc = (node_b2 + node_bres).reshape(1, NODE_DIM)
    new_x = _node_tc(
        x, aggp[:, :N_NODES, :],
        node_W1[:NODE_DIM], node_W1[NODE_DIM:],
        node_b1.reshape(1, HIDDEN),
        node_W2, node_Wres[:NODE_DIM], node_Wres[NODE_DIM:], nbc,
        node_gamma.reshape(1, NODE_DIM), node_beta.reshape(1, NODE_DIM))
    return (new_x, new_edge)


# speedup vs baseline: 1.2742x; 1.0154x over previous
import functools

import jax
import jax.numpy as jnp
from jax import lax
from jax.experimental import pallas as pl
from jax.experimental.pallas import tpu as pltpu
from jax.experimental.pallas import tpu_sc as plsc

NODE_DIM = 128
EDGE_DIM = 16
HIDDEN = 128
N_NODES = 10000
N_EDGES = 320000
TDIM = HIDDEN + EDGE_DIM
NPAD = 10240
NEP = 327680
C = 32
GC = 80
GATHER_S0 = 128


def _sc_geometry():
    try:
        info = plsc.get_sparse_core_info()
        return int(info.num_cores), int(info.num_subcores)
    except Exception:
        return 2, 16


def _tables_tc(x, wcat_s, wcat_d):
    blk = 1000

    def body(x_ref, ws_ref, wd_ref, ts_ref, td_ref, ts2_ref, td2_ref):
        xb = x_ref[...]
        ts = jnp.dot(xb, ws_ref[...], preferred_element_type=jnp.float32)
        td = jnp.dot(xb, wd_ref[...], preferred_element_type=jnp.float32)
        ts_ref[...] = ts
        td_ref[...] = td
        ts2_ref[...] = ts
        td2_ref[...] = td

    full = lambda i: (i, 0)
    return pl.pallas_call(
        body,
        grid=(N_NODES // blk,),
        in_specs=[
            pl.BlockSpec((blk, NODE_DIM), full),
            pl.BlockSpec((NODE_DIM, TDIM), lambda i: (0, 0)),
            pl.BlockSpec((NODE_DIM, TDIM), lambda i: (0, 0)),
        ],
        out_specs=[
            pl.BlockSpec((blk, TDIM), full),
            pl.BlockSpec((blk, TDIM), full),
            pl.BlockSpec((blk, TDIM), full),
            pl.BlockSpec((blk, TDIM), full),
        ],
        out_shape=[
            jax.ShapeDtypeStruct((N_NODES, TDIM), jnp.float32),
            jax.ShapeDtypeStruct((N_NODES, TDIM), jnp.float32),
            jax.ShapeDtypeStruct((N_NODES, TDIM), jnp.float32),
            jax.ShapeDtypeStruct((N_NODES, TDIM), jnp.float32),
        ],
    )(x, wcat_s, wcat_d)


def _build_gather(nc, ns):
    nw = nc * ns
    spw = NEP // (GC * nw) * nc
    if nc == 2:
        s_per = (GATHER_S0, spw - GATHER_S0)
    else:
        s_per = (spw,)
    smax = max(s_per)
    mesh = plsc.VectorSubcoreMesh(core_axis_name="c", subcore_axis_name="s",
                                  num_cores=nc, num_subcores=ns)

    @functools.partial(
        pl.kernel,
        out_type=(
            jax.ShapeDtypeStruct((NEP, HIDDEN), jnp.float32),
            jax.ShapeDtypeStruct((NEP, HIDDEN), jnp.float32),
            jax.ShapeDtypeStruct((NEP // 8, HIDDEN), jnp.float32),
        ),
        mesh=mesh,
        compiler_params=pltpu.CompilerParams(use_tc_tiling_on_sc=False),
        scratch_types=[
            pltpu.VMEM((smax, GC), jnp.int32),
            pltpu.VMEM((smax, GC), jnp.int32),
            pltpu.VMEM((3, GC, TDIM), jnp.float32),
            pltpu.VMEM((3, GC, TDIM), jnp.float32),
            pltpu.VMEM((3, GC // 8, HIDDEN), jnp.float32),
            pltpu.SemaphoreType.DMA,
            pltpu.SemaphoreType.DMA,
            pltpu.SemaphoreType.DMA,
            pltpu.SemaphoreType.DMA,
            pltpu.SemaphoreType.DMA,
            pltpu.SemaphoreType.DMA,
        ],
    )
    def gather_k(ts_hbm, td_hbm, ts2_hbm, td2_hbm, s2_hbm, d2_hbm,
                 gms_hbm, gmd_hbm, grp_hbm,
                 sidx, didx, sbuf, dbuf, rbuf,
                 gsem0, gsem1, gsem2, wsem0, wsem1, wsem2):
        cid = lax.axis_index("c")
        sid = lax.axis_index("s")

        if nc == 2:
            sbase = jnp.where(cid == 0, sid * s_per[0],
                              ns * s_per[0] + sid * s_per[1])
            my_s = jnp.where(cid == 0, s_per[0], s_per[1])

            @pl.when(cid == 0)
            def _():
                pltpu.sync_copy(s2_hbm.at[pl.ds(sid * s_per[0], s_per[0])],
                                sidx.at[pl.ds(0, s_per[0])])
                pltpu.sync_copy(d2_hbm.at[pl.ds(sid * s_per[0], s_per[0])],
                                didx.at[pl.ds(0, s_per[0])])

            @pl.when(cid == 1)
            def _():
                b = ns * s_per[0] + sid * s_per[1]
                pltpu.sync_copy(s2_hbm.at[pl.ds(b, s_per[1])],
                                sidx.at[pl.ds(0, s_per[1])])
                pltpu.sync_copy(d2_hbm.at[pl.ds(b, s_per[1])],
                                didx.at[pl.ds(0, s_per[1])])
        else:
            sbase = sid * s_per[0]
            my_s = s_per[0]
            pltpu.sync_copy(s2_hbm.at[pl.ds(sbase, s_per[0])], sidx)
            pltpu.sync_copy(d2_hbm.at[pl.ds(sbase, s_per[0])], didx)

        gsems = (gsem0, gsem1, gsem2)
        wsems = (wsem0, wsem1, wsem2)

        def fire(s, p):
            @pl.when(cid == 0)
            def _():
                pltpu.async_copy(ts_hbm.at[sidx.at[s]], sbuf.at[p], gsems[p])
                pltpu.async_copy(td_hbm.at[didx.at[s]], dbuf.at[p], gsems[p])

            @pl.when(cid == 1)
            def _():
                pltpu.async_copy(ts2_hbm.at[sidx.at[s]], sbuf.at[p], gsems[p])
                pltpu.async_copy(td2_hbm.at[didx.at[s]], dbuf.at[p], gsems[p])

        def drain_gather(s, p):
            pltpu.make_async_copy(ts_hbm.at[sidx.at[s]], sbuf.at[p],
                                  gsems[p]).wait()
            pltpu.make_async_copy(td_hbm.at[didx.at[s]], dbuf.at[p],
                                  gsems[p]).wait()

        def res_sum(p):
            def rbody(r2, carry):
                for q in range(8):
                    rbuf[p, r2, pl.ds(q * EDGE_DIM, EDGE_DIM)] = (
                        sbuf[p, r2 * 8 + q, pl.ds(HIDDEN, EDGE_DIM)]
                        + dbuf[p, r2 * 8 + q, pl.ds(HIDDEN, EDGE_DIM)])
                return carry

            lax.fori_loop(0, GC // 8, rbody, 0)

        def _wcopies(g, p):
            r = pl.ds(g * GC, GC)
            rp = pl.ds(g * (GC // 8), GC // 8)
            return (
                (sbuf.at[p, :, pl.ds(0, HIDDEN)], gms_hbm.at[r], wsems[p]),
                (dbuf.at[p, :, pl.ds(0, HIDDEN)], gmd_hbm.at[r], wsems[p]),
                (rbuf.at[p], grp_hbm.at[rp], wsems[p]),
            )

        def write(g, p):
            for a, b, sem in _wcopies(g, p):
                pltpu.async_copy(a, b, sem)

        def drain_write(g, p):
            for a, b, sem in _wcopies(g, p):
                pltpu.make_async_copy(a, b, sem).wait()

        fire(0, 0)

        @pl.when(my_s > 1)
        def _():
            fire(1, 1)

        @pl.when(my_s > 2)
        def _():
            fire(2, 2)

        def body(t, carry):
            for p in range(3):
                s = 3 * t + p

                @pl.when(s < my_s)
                def _():
                    drain_gather(s, p)
                    res_sum(p)
                    write(sbase + s, p)
                    drain_write(sbase + s, p)
                    nxt = s + 3

                    @pl.when(nxt < my_s)
                    def _():
                        fire(nxt, p)

            return carry

        lax.fori_loop(0, (smax + 2) // 3, body, 0)

    return gather_k


def _unpack16(gp, blk):
    parts = jnp.stack(
        [gp[:, q * EDGE_DIM:(q + 1) * EDGE_DIM] for q in range(8)], axis=1)
    return jnp.reshape(parts, (blk, EDGE_DIM))


def _pack16(ne, blk):
    r = jnp.reshape(ne, (blk // 8, 8, EDGE_DIM))
    return jnp.concatenate([r[:, q, :] for q in range(8)], axis=1)


def _edge_tc(gms, gmd, grp, ea, w1e, b1, w2, wre, bc, gamma, beta):
    blk = 2048

    def body(gms_ref, gmd_ref, grp_ref, ea_ref, w1e_ref, b1_ref,
             w2_ref, wre_ref, bc_ref, g_ref, be_ref, o_ref, op_ref):
        eab = ea_ref[...].astype(jnp.bfloat16)
        h = (gms_ref[...] + gmd_ref[...]
             + jnp.dot(eab, w1e_ref[...], preferred_element_type=jnp.float32)
             + b1_ref[...])
        h = h * (1.0 / (1.0 + jnp.exp(-h)))
        o = (jnp.dot(h.astype(jnp.bfloat16), w2_ref[...],
                     preferred_element_type=jnp.float32)
             + jnp.dot(eab, wre_ref[...], preferred_element_type=jnp.float32)
             + _unpack16(grp_ref[...], blk) + bc_ref[...])
        m = jnp.mean(o, axis=1, keepdims=True)
        cde = o - m
        v = jnp.mean(cde * cde, axis=1, keepdims=True)
        ne = cde * lax.rsqrt(v + 1e-5) * g_ref[...] + be_ref[...]
        o_ref[...] = ne
        op_ref[...] = _pack16(ne, blk)

    return pl.pallas_call(
        body,
        grid=(-(-N_EDGES // blk),),
        in_specs=[
            pl.BlockSpec((blk, HIDDEN), lambda i: (i, 0)),
            pl.BlockSpec((blk, HIDDEN), lambda i: (i, 0)),
            pl.BlockSpec((blk // 8, HIDDEN), lambda i: (i, 0)),
            pl.BlockSpec((blk, EDGE_DIM), lambda i: (i, 0)),
            pl.BlockSpec((EDGE_DIM, HIDDEN), lambda i: (0, 0)),
            pl.BlockSpec((1, HIDDEN), lambda i: (0, 0)),
            pl.BlockSpec((HIDDEN, EDGE_DIM), lambda i: (0, 0)),
            pl.BlockSpec((EDGE_DIM, EDGE_DIM), lambda i: (0, 0)),
            pl.BlockSpec((1, EDGE_DIM), lambda i: (0, 0)),
            pl.BlockSpec((1, EDGE_DIM), lambda i: (0, 0)),
            pl.BlockSpec((1, EDGE_DIM), lambda i: (0, 0)),
        ],
        out_specs=[
            pl.BlockSpec((blk, EDGE_DIM), lambda i: (i, 0)),
            pl.BlockSpec((blk // 8, HIDDEN), lambda i: (i, 0)),
        ],
        out_shape=[
            jax.ShapeDtypeStruct((N_EDGES, EDGE_DIM), jnp.float32),
            jax.ShapeDtypeStruct((NEP // 8, HIDDEN), jnp.float32),
        ],
    )(gms, gmd, grp, ea, w1e, b1, w2, wre, bc, gamma, beta)


def _build_scatter(nc, ns):
    nw = nc * ns
    pw = NEP // nw
    c2 = 64
    ki = 40
    chunk = ki * c2
    outer = pw // chunk
    rows_per = NPAD // ns
    mesh = plsc.VectorSubcoreMesh(core_axis_name="c", subcore_axis_name="s",
                                  num_cores=nc, num_subcores=ns)

    @functools.partial(
        pl.kernel,
        out_type=jax.ShapeDtypeStruct((nc, NPAD, EDGE_DIM), jnp.float32),
        mesh=mesh,
        compiler_params=pltpu.CompilerParams(use_tc_tiling_on_sc=False),
        scratch_types=[
            pltpu.VMEM((ki, c2), jnp.int32),
            pltpu.VMEM((chunk // 8, HIDDEN), jnp.float32),
            pltpu.VMEM((chunk, EDGE_DIM), jnp.float32),
            pltpu.VMEM((rows_per, EDGE_DIM), jnp.float32),
            pltpu.VMEM_SHARED((NPAD, EDGE_DIM), jnp.float32),
        ],
    )
    def scatter_k(nep_hbm, d2_hbm, out_hbm, idxb, rowsb128, rowsb, bounce,
                  aggsh):
        cid = lax.axis_index("c")
        sid = lax.axis_index("s")
        z = jnp.zeros((16,), jnp.float32)

        def zbody(i, carry):
            bounce[i, :] = z
            return carry

        lax.fori_loop(0, rows_per, zbody, 0)
        pltpu.sync_copy(bounce, aggsh.at[pl.ds(sid * rows_per, rows_per)])
        plsc.subcore_barrier()

        wid = cid * ns + sid
        e0 = wid * pw
        row0 = e0 // c2
        prow0 = e0 // 8
        pchunk = chunk // 8

        def body(t, carry):
            pltpu.sync_copy(d2_hbm.at[pl.ds(row0 + t * ki, ki)], idxb)
            pltpu.sync_copy(nep_hbm.at[pl.ds(prow0 + t * pchunk, pchunk)],
                            rowsb128)

            def rbody(r, carry2):
                for q in range(8):
                    rowsb[r * 8 + q, :] = rowsb128[r, pl.ds(q * EDGE_DIM,
                                                            EDGE_DIM)]
                return carry2

            lax.fori_loop(0, pchunk, rbody, 0)
            for j in range(ki):
                pltpu.sync_copy(rowsb.at[pl.ds(j * c2, c2)],
                                aggsh.at[idxb.at[j]], add=True)
            return carry

        lax.fori_loop(0, outer, body, 0)
        plsc.subcore_barrier()
        pltpu.sync_copy(aggsh.at[pl.ds(sid * rows_per, rows_per)], bounce)
        pltpu.sync_copy(bounce, out_hbm.at[cid, pl.ds(sid * rows_per, rows_per)])

    return scatter_k


def _node_tc(x, aggp, w1x, w1a, b1, w2, wrx, wra, bc, gamma, beta):
    blk = 1000
    nc = aggp.shape[0]

    def body(x_ref, ap_ref, w1x_ref, w1a_ref, b1_ref, w2_ref, wrx_ref,
             wra_ref, bc_ref, g_ref, be_ref, o_ref):
        xb = x_ref[...]
        a = jnp.sum(ap_ref[...], axis=0)
        h = (jnp.dot(xb, w1x_ref[...], preferred_element_type=jnp.float32)
             + jnp.dot(a, w1a_ref[...], preferred_element_type=jnp.float32)
             + b1_ref[...])
        h = h * (1.0 / (1.0 + jnp.exp(-h)))
        o = (jnp.dot(h, w2_ref[...], preferred_element_type=jnp.float32)
             + jnp.dot(xb, wrx_ref[...], preferred_element_type=jnp.float32)
             + jnp.dot(a, wra_ref[...], preferred_element_type=jnp.float32)
             + bc_ref[...])
        m = jnp.mean(o, axis=1, keepdims=True)
        cde = o - m
        v = jnp.mean(cde * cde, axis=1, keepdims=True)
        o_ref[...] = cde * lax.rsqrt(v + 1e-5) * g_ref[...] + be_ref[...]

    return pl.pallas_call(
        body,
        grid=(N_NODES // blk,),
        in_specs=[
            pl.BlockSpec((blk, NODE_DIM), lambda i: (i, 0)),
            pl.BlockSpec((nc, blk, EDGE_DIM), lambda i: (0, i, 0)),
            pl.BlockSpec((NODE_DIM, HIDDEN), lambda i: (0, 0)),
            pl.BlockSpec((EDGE_DIM, HIDDEN), lambda i: (0, 0)),
            pl.BlockSpec((1, HIDDEN), lambda i: (0, 0)),
            pl.BlockSpec((HIDDEN, NODE_DIM), lambda i: (0, 0)),
            pl.BlockSpec((NODE_DIM, NODE_DIM), lambda i: (0, 0)),
            pl.BlockSpec((EDGE_DIM, NODE_DIM), lambda i: (0, 0)),
            pl.BlockSpec((1, NODE_DIM), lambda i: (0, 0)),
            pl.BlockSpec((1, NODE_DIM), lambda i: (0, 0)),
            pl.BlockSpec((1, NODE_DIM), lambda i: (0, 0)),
        ],
        out_specs=pl.BlockSpec((blk, NODE_DIM), lambda i: (i, 0)),
        out_shape=jax.ShapeDtypeStruct((N_NODES, NODE_DIM), jnp.float32),
    )(x, aggp, w1x, w1a, b1, w2, wrx, wra, bc, gamma, beta)


def kernel(x, edge_attr, edge_index,
           edge_W1, edge_b1, edge_W2, edge_b2, edge_Wres, edge_bres,
           edge_gamma, edge_beta,
           node_W1, node_b1, node_W2, node_b2, node_Wres, node_bres,
           node_gamma, node_beta):
    nc, ns = _sc_geometry()

    npad_e = NEP - N_EDGES
    src_p = jnp.concatenate(
        [edge_index[0], jnp.zeros((npad_e,), jnp.int32)])
    dst_p = jnp.concatenate(
        [edge_index[1], jnp.full((npad_e,), N_NODES, jnp.int32)])
    src2g = src_p.reshape(NEP // GC, GC)
    dst2g = dst_p.reshape(NEP // GC, GC)
    dst2 = dst_p.reshape(NEP // 64, 64)
    ea_p = jnp.concatenate(
        [edge_attr, jnp.zeros((npad_e, EDGE_DIM), jnp.float32)])

    w1e = edge_W1[:EDGE_DIM]
    wcat_s = jnp.concatenate(
        [edge_W1[EDGE_DIM:EDGE_DIM + NODE_DIM],
         edge_Wres[EDGE_DIM:EDGE_DIM + NODE_DIM]], axis=1)
    wcat_d = jnp.concatenate(
        [edge_W1[EDGE_DIM + NODE_DIM:],
         edge_Wres[EDGE_DIM + NODE_DIM:]], axis=1)
    wre = edge_Wres[:EDGE_DIM]
    ebc = (edge_b2 + edge_bres).reshape(1, EDGE_DIM)

    tsrc, tdst, tsrc2, tdst2 = _tables_tc(x, wcat_s, wcat_d)
    gms, gmd, grp = _build_gather(nc, ns)(tsrc, tdst, tsrc2, tdst2,
                                          src2g, dst2g)
    new_edge, nep = _edge_tc(gms, gmd, grp, ea_p,
                             w1e.astype(jnp.bfloat16),
                             edge_b1.reshape(1, HIDDEN),
                             edge_W2.astype(jnp.bfloat16),
                             wre.astype(jnp.bfloat16), ebc,
                             edge_gamma.reshape(1, EDGE_DIM),
                             edge_beta.reshape(1, EDGE_DIM))
    aggp = _build_scatter(nc, ns)(nep, dst2)

    nbc = (node_b2 + node_bres).reshape(1, NODE_DIM)
    new_x = _node_tc(
        x, aggp[:, :N_NODES, :],
        node_W1[:NODE_DIM], node_W1[NODE_DIM:],
        node_b1.reshape(1, HIDDEN),
        node_W2, node_Wres[:NODE_DIM], node_Wres[NODE_DIM:], nbc,
        node_gamma.reshape(1, NODE_DIM), node_beta.reshape(1, NODE_DIM))
    return (new_x, new_edge)


# speedup vs baseline: 1.2756x; 1.0011x over previous
import functools

import jax
import jax.numpy as jnp
from jax import lax
from jax.experimental import pallas as pl
from jax.experimental.pallas import tpu as pltpu
from jax.experimental.pallas import tpu_sc as plsc

NODE_DIM = 128
EDGE_DIM = 16
HIDDEN = 128
N_NODES = 10000
N_EDGES = 320000
TDIM = HIDDEN + EDGE_DIM
NPAD = 10240
NEP = 327680
C = 32
GC = 80
GATHER_S0 = 128


def _sc_geometry():
    try:
        info = plsc.get_sparse_core_info()
        return int(info.num_cores), int(info.num_subcores)
    except Exception:
        return 2, 16


def _tables_tc(x, wcat_s, wcat_d):
    blk = 1000

    def body(x_ref, ws_ref, wd_ref, ts_ref, td_ref, ts2_ref, td2_ref):
        xb = x_ref[...]
        ts = jnp.dot(xb, ws_ref[...], preferred_element_type=jnp.float32)
        td = jnp.dot(xb, wd_ref[...], preferred_element_type=jnp.float32)
        ts_ref[...] = ts
        td_ref[...] = td
        ts2_ref[...] = ts
        td2_ref[...] = td

    full = lambda i: (i, 0)
    return pl.pallas_call(
        body,
        grid=(N_NODES // blk,),
        in_specs=[
            pl.BlockSpec((blk, NODE_DIM), full),
            pl.BlockSpec((NODE_DIM, TDIM), lambda i: (0, 0)),
            pl.BlockSpec((NODE_DIM, TDIM), lambda i: (0, 0)),
        ],
        out_specs=[
            pl.BlockSpec((blk, TDIM), full),
            pl.BlockSpec((blk, TDIM), full),
            pl.BlockSpec((blk, TDIM), full),
            pl.BlockSpec((blk, TDIM), full),
        ],
        out_shape=[
            jax.ShapeDtypeStruct((N_NODES, TDIM), jnp.float32),
            jax.ShapeDtypeStruct((N_NODES, TDIM), jnp.float32),
            jax.ShapeDtypeStruct((N_NODES, TDIM), jnp.float32),
            jax.ShapeDtypeStruct((N_NODES, TDIM), jnp.float32),
        ],
    )(x, wcat_s, wcat_d)


def _build_gather(nc, ns):
    nw = nc * ns
    spw = NEP // (GC * nw) * nc
    if nc == 2:
        s_per = (GATHER_S0, spw - GATHER_S0)
    else:
        s_per = (spw,)
    smax = max(s_per)
    mesh = plsc.VectorSubcoreMesh(core_axis_name="c", subcore_axis_name="s",
                                  num_cores=nc, num_subcores=ns)

    @functools.partial(
        pl.kernel,
        out_type=(
            jax.ShapeDtypeStruct((NEP, HIDDEN), jnp.float32),
            jax.ShapeDtypeStruct((NEP, HIDDEN), jnp.float32),
            jax.ShapeDtypeStruct((NEP // 8, HIDDEN), jnp.float32),
        ),
        mesh=mesh,
        compiler_params=pltpu.CompilerParams(use_tc_tiling_on_sc=False),
        scratch_types=[
            pltpu.VMEM((smax, GC), jnp.int32),
            pltpu.VMEM((smax, GC), jnp.int32),
            pltpu.VMEM((4, GC, TDIM), jnp.float32),
            pltpu.VMEM((4, GC, TDIM), jnp.float32),
            pltpu.VMEM((4, GC // 8, HIDDEN), jnp.float32),
            pltpu.SemaphoreType.DMA,
            pltpu.SemaphoreType.DMA,
            pltpu.SemaphoreType.DMA,
            pltpu.SemaphoreType.DMA,
            pltpu.SemaphoreType.DMA,
            pltpu.SemaphoreType.DMA,
            pltpu.SemaphoreType.DMA,
            pltpu.SemaphoreType.DMA,
        ],
    )
    def gather_k(ts_hbm, td_hbm, ts2_hbm, td2_hbm, s2_hbm, d2_hbm,
                 gms_hbm, gmd_hbm, grp_hbm,
                 sidx, didx, sbuf, dbuf, rbuf,
                 gsem0, gsem1, gsem2, gsem3, wsem0, wsem1, wsem2, wsem3):
        cid = lax.axis_index("c")
        sid = lax.axis_index("s")

        if nc == 2:
            sbase = jnp.where(cid == 0, sid * s_per[0],
                              ns * s_per[0] + sid * s_per[1])
            my_s = jnp.where(cid == 0, s_per[0], s_per[1])

            @pl.when(cid == 0)
            def _():
                pltpu.sync_copy(s2_hbm.at[pl.ds(sid * s_per[0], s_per[0])],
                                sidx.at[pl.ds(0, s_per[0])])
                pltpu.sync_copy(d2_hbm.at[pl.ds(sid * s_per[0], s_per[0])],
                                didx.at[pl.ds(0, s_per[0])])

            @pl.when(cid == 1)
            def _():
                b = ns * s_per[0] + sid * s_per[1]
                pltpu.sync_copy(s2_hbm.at[pl.ds(b, s_per[1])],
                                sidx.at[pl.ds(0, s_per[1])])
                pltpu.sync_copy(d2_hbm.at[pl.ds(b, s_per[1])],
                                didx.at[pl.ds(0, s_per[1])])
        else:
            sbase = sid * s_per[0]
            my_s = s_per[0]
            pltpu.sync_copy(s2_hbm.at[pl.ds(sbase, s_per[0])], sidx)
            pltpu.sync_copy(d2_hbm.at[pl.ds(sbase, s_per[0])], didx)

        gsems = (gsem0, gsem1, gsem2, gsem3)
        wsems = (wsem0, wsem1, wsem2, wsem3)

        def fire(s, p):
            @pl.when(cid == 0)
            def _():
                pltpu.async_copy(ts_hbm.at[sidx.at[s]], sbuf.at[p], gsems[p])
                pltpu.async_copy(td_hbm.at[didx.at[s]], dbuf.at[p], gsems[p])

            @pl.when(cid == 1)
            def _():
                pltpu.async_copy(ts2_hbm.at[sidx.at[s]], sbuf.at[p], gsems[p])
                pltpu.async_copy(td2_hbm.at[didx.at[s]], dbuf.at[p], gsems[p])

        def drain_gather(s, p):
            pltpu.make_async_copy(ts_hbm.at[sidx.at[s]], sbuf.at[p],
                                  gsems[p]).wait()
            pltpu.make_async_copy(td_hbm.at[didx.at[s]], dbuf.at[p],
                                  gsems[p]).wait()

        def res_sum(p):
            def rbody(r2, carry):
                for q in range(8):
                    rbuf[p, r2, pl.ds(q * EDGE_DIM, EDGE_DIM)] = (
                        sbuf[p, r2 * 8 + q, pl.ds(HIDDEN, EDGE_DIM)]
                        + dbuf[p, r2 * 8 + q, pl.ds(HIDDEN, EDGE_DIM)])
                return carry

            lax.fori_loop(0, GC // 8, rbody, 0)

        def _wcopies(g, p):
            r = pl.ds(g * GC, GC)
            rp = pl.ds(g * (GC // 8), GC // 8)
            return (
                (sbuf.at[p, :, pl.ds(0, HIDDEN)], gms_hbm.at[r], wsems[p]),
                (dbuf.at[p, :, pl.ds(0, HIDDEN)], gmd_hbm.at[r], wsems[p]),
                (rbuf.at[p], grp_hbm.at[rp], wsems[p]),
            )

        def write(g, p):
            for a, b, sem in _wcopies(g, p):
                pltpu.async_copy(a, b, sem)

        def drain_write(g, p):
            for a, b, sem in _wcopies(g, p):
                pltpu.make_async_copy(a, b, sem).wait()

        fire(0, 0)

        @pl.when(my_s > 1)
        def _():
            fire(1, 1)

        @pl.when(my_s > 2)
        def _():
            fire(2, 2)

        @pl.when(my_s > 3)
        def _():
            fire(3, 3)

        def body(t, carry):
            for p in range(4):
                s = 4 * t + p

                @pl.when(s < my_s)
                def _():
                    drain_gather(s, p)
                    res_sum(p)
                    write(sbase + s, p)
                    drain_write(sbase + s, p)
                    nxt = s + 4

                    @pl.when(nxt < my_s)
                    def _():
                        fire(nxt, p)

            return carry

        lax.fori_loop(0, (smax + 3) // 4, body, 0)

    return gather_k


def _unpack16(gp, blk):
    parts = jnp.stack(
        [gp[:, q * EDGE_DIM:(q + 1) * EDGE_DIM] for q in range(8)], axis=1)
    return jnp.reshape(parts, (blk, EDGE_DIM))


def _pack16(ne, blk):
    r = jnp.reshape(ne, (blk // 8, 8, EDGE_DIM))
    return jnp.concatenate([r[:, q, :] for q in range(8)], axis=1)


def _edge_tc(gms, gmd, grp, ea, w1e, b1, w2, wre, bc, gamma, beta):
    blk = 2048

    def body(gms_ref, gmd_ref, grp_ref, ea_ref, w1e_ref, b1_ref,
             w2_ref, wre_ref, bc_ref, g_ref, be_ref, o_ref, op_ref):
        eab = ea_ref[...].astype(jnp.bfloat16)
        h = (gms_ref[...] + gmd_ref[...]
             + jnp.dot(eab, w1e_ref[...], preferred_element_type=jnp.float32)
             + b1_ref[...])
        h = h * (1.0 / (1.0 + jnp.exp(-h)))
        o = (jnp.dot(h.astype(jnp.bfloat16), w2_ref[...],
                     preferred_element_type=jnp.float32)
             + jnp.dot(eab, wre_ref[...], preferred_element_type=jnp.float32)
             + _unpack16(grp_ref[...], blk) + bc_ref[...])
        m = jnp.mean(o, axis=1, keepdims=True)
        cde = o - m
        v = jnp.mean(cde * cde, axis=1, keepdims=True)
        ne = cde * lax.rsqrt(v + 1e-5) * g_ref[...] + be_ref[...]
        o_ref[...] = ne
        op_ref[...] = _pack16(ne, blk)

    return pl.pallas_call(
        body,
        grid=(-(-N_EDGES // blk),),
        in_specs=[
            pl.BlockSpec((blk, HIDDEN), lambda i: (i, 0)),
            pl.BlockSpec((blk, HIDDEN), lambda i: (i, 0)),
            pl.BlockSpec((blk // 8, HIDDEN), lambda i: (i, 0)),
            pl.BlockSpec((blk, EDGE_DIM), lambda i: (i, 0)),
            pl.BlockSpec((EDGE_DIM, HIDDEN), lambda i: (0, 0)),
            pl.BlockSpec((1, HIDDEN), lambda i: (0, 0)),
            pl.BlockSpec((HIDDEN, EDGE_DIM), lambda i: (0, 0)),
            pl.BlockSpec((EDGE_DIM, EDGE_DIM), lambda i: (0, 0)),
            pl.BlockSpec((1, EDGE_DIM), lambda i: (0, 0)),
            pl.BlockSpec((1, EDGE_DIM), lambda i: (0, 0)),
            pl.BlockSpec((1, EDGE_DIM), lambda i: (0, 0)),
        ],
        out_specs=[
            pl.BlockSpec((blk, EDGE_DIM), lambda i: (i, 0)),
            pl.BlockSpec((blk // 8, HIDDEN), lambda i: (i, 0)),
        ],
        out_shape=[
            jax.ShapeDtypeStruct((N_EDGES, EDGE_DIM), jnp.float32),
            jax.ShapeDtypeStruct((NEP // 8, HIDDEN), jnp.float32),
        ],
    )(gms, gmd, grp, ea, w1e, b1, w2, wre, bc, gamma, beta)


def _build_scatter(nc, ns):
    nw = nc * ns
    pw = NEP // nw
    c2 = 64
    ki = 40
    chunk = ki * c2
    outer = pw // chunk
    rows_per = NPAD // ns
    mesh = plsc.VectorSubcoreMesh(core_axis_name="c", subcore_axis_name="s",
                                  num_cores=nc, num_subcores=ns)

    @functools.partial(
        pl.kernel,
        out_type=jax.ShapeDtypeStruct((nc, NPAD, EDGE_DIM), jnp.float32),
        mesh=mesh,
        compiler_params=pltpu.CompilerParams(use_tc_tiling_on_sc=False),
        scratch_types=[
            pltpu.VMEM((ki, c2), jnp.int32),
            pltpu.VMEM((chunk // 8, HIDDEN), jnp.float32),
            pltpu.VMEM((chunk, EDGE_DIM), jnp.float32),
            pltpu.VMEM((rows_per, EDGE_DIM), jnp.float32),
            pltpu.VMEM_SHARED((NPAD, EDGE_DIM), jnp.float32),
        ],
    )
    def scatter_k(nep_hbm, d2_hbm, out_hbm, idxb, rowsb128, rowsb, bounce,
                  aggsh):
        cid = lax.axis_index("c")
        sid = lax.axis_index("s")
        z = jnp.zeros((16,), jnp.float32)

        def zbody(i, carry):
            bounce[i, :] = z
            return carry

        lax.fori_loop(0, rows_per, zbody, 0)
        pltpu.sync_copy(bounce, aggsh.at[pl.ds(sid * rows_per, rows_per)])
        plsc.subcore_barrier()

        wid = cid * ns + sid
        e0 = wid * pw
        row0 = e0 // c2
        prow0 = e0 // 8
        pchunk = chunk // 8

        def body(t, carry):
            pltpu.sync_copy(d2_hbm.at[pl.ds(row0 + t * ki, ki)], idxb)
            pltpu.sync_copy(nep_hbm.at[pl.ds(prow0 + t * pchunk, pchunk)],
                            rowsb128)

            def rbody(r, carry2):
                for q in range(8):
                    rowsb[r * 8 + q, :] = rowsb128[r, pl.ds(q * EDGE_DIM,
                                                            EDGE_DIM)]
                return carry2

            lax.fori_loop(0, pchunk, rbody, 0)
            for j in range(ki):
                pltpu.sync_copy(rowsb.at[pl.ds(j * c2, c2)],
                                aggsh.at[idxb.at[j]], add=True)
            return carry

        lax.fori_loop(0, outer, body, 0)
        plsc.subcore_barrier()
        pltpu.sync_copy(aggsh.at[pl.ds(sid * rows_per, rows_per)], bounce)
        pltpu.sync_copy(bounce, out_hbm.at[cid, pl.ds(sid * rows_per, rows_per)])

    return scatter_k


def _node_tc(x, aggp, w1x, w1a, b1, w2, wrx, wra, bc, gamma, beta):
    blk = 1000
    nc = aggp.shape[0]

    def body(x_ref, ap_ref, w1x_ref, w1a_ref, b1_ref, w2_ref, wrx_ref,
             wra_ref, bc_ref, g_ref, be_ref, o_ref):
        xb = x_ref[...]
        a = jnp.sum(ap_ref[...], axis=0)
        h = (jnp.dot(xb, w1x_ref[...], preferred_element_type=jnp.float32)
             + jnp.dot(a, w1a_ref[...], preferred_element_type=jnp.float32)
             + b1_ref[...])
        h = h * (1.0 / (1.0 + jnp.exp(-h)))
        o = (jnp.dot(h, w2_ref[...], preferred_element_type=jnp.float32)
             + jnp.dot(xb, wrx_ref[...], preferred_element_type=jnp.float32)
             + jnp.dot(a, wra_ref[...], preferred_element_type=jnp.float32)
             + bc_ref[...])
        m = jnp.mean(o, axis=1, keepdims=True)
        cde = o - m
        v = jnp.mean(cde * cde, axis=1, keepdims=True)
        o_ref[...] = cde * lax.rsqrt(v + 1e-5) * g_ref[...] + be_ref[...]

    return pl.pallas_call(
        body,
        grid=(N_NODES // blk,),
        in_specs=[
            pl.BlockSpec((blk, NODE_DIM), lambda i: (i, 0)),
            pl.BlockSpec((nc, blk, EDGE_DIM), lambda i: (0, i, 0)),
            pl.BlockSpec((NODE_DIM, HIDDEN), lambda i: (0, 0)),
            pl.BlockSpec((EDGE_DIM, HIDDEN), lambda i: (0, 0)),
            pl.BlockSpec((1, HIDDEN), lambda i: (0, 0)),
            pl.BlockSpec((HIDDEN, NODE_DIM), lambda i: (0, 0)),
            pl.BlockSpec((NODE_DIM, NODE_DIM), lambda i: (0, 0)),
            pl.BlockSpec((EDGE_DIM, NODE_DIM), lambda i: (0, 0)),
            pl.BlockSpec((1, NODE_DIM), lambda i: (0, 0)),
            pl.BlockSpec((1, NODE_DIM), lambda i: (0, 0)),
            pl.BlockSpec((1, NODE_DIM), lambda i: (0, 0)),
        ],
        out_specs=pl.BlockSpec((blk, NODE_DIM), lambda i: (i, 0)),
        out_shape=jax.ShapeDtypeStruct((N_NODES, NODE_DIM), jnp.float32),
    )(x, aggp, w1x, w1a, b1, w2, wrx, wra, bc, gamma, beta)


def kernel(x, edge_attr, edge_index,
           edge_W1, edge_b1, edge_W2, edge_b2, edge_Wres, edge_bres,
           edge_gamma, edge_beta,
           node_W1, node_b1, node_W2, node_b2, node_Wres, node_bres,
           node_gamma, node_beta):
    nc, ns = _sc_geometry()

    npad_e = NEP - N_EDGES
    src_p = jnp.concatenate(
        [edge_index[0], jnp.zeros((npad_e,), jnp.int32)])
    dst_p = jnp.concatenate(
        [edge_index[1], jnp.full((npad_e,), N_NODES, jnp.int32)])
    src2g = src_p.reshape(NEP // GC, GC)
    dst2g = dst_p.reshape(NEP // GC, GC)
    dst2 = dst_p.reshape(NEP // 64, 64)
    ea_p = jnp.concatenate(
        [edge_attr, jnp.zeros((npad_e, EDGE_DIM), jnp.float32)])

    w1e = edge_W1[:EDGE_DIM]
    wcat_s = jnp.concatenate(
        [edge_W1[EDGE_DIM:EDGE_DIM + NODE_DIM],
         edge_Wres[EDGE_DIM:EDGE_DIM + NODE_DIM]], axis=1)
    wcat_d = jnp.concatenate(
        [edge_W1[EDGE_DIM + NODE_DIM:],
         edge_Wres[EDGE_DIM + NODE_DIM:]], axis=1)
    wre = edge_Wres[:EDGE_DIM]
    ebc = (edge_b2 + edge_bres).reshape(1, EDGE_DIM)

    tsrc, tdst, tsrc2, tdst2 = _tables_tc(x, wcat_s, wcat_d)
    gms, gmd, grp = _build_gather(nc, ns)(tsrc, tdst, tsrc2, tdst2,
                                          src2g, dst2g)
    new_edge, nep = _edge_tc(gms, gmd, grp, ea_p,
                             w1e.astype(jnp.bfloat16),
                             edge_b1.reshape(1, HIDDEN),
                             edge_W2.astype(jnp.bfloat16),
                             wre.astype(jnp.bfloat16), ebc,
                             edge_gamma.reshape(1, EDGE_DIM),
                             edge_beta.reshape(1, EDGE_DIM))
    aggp = _build_scatter(nc, ns)(nep, dst2)

    nbc = (node_b2 + node_bres).reshape(1, NODE_DIM)
    new_x = _node_tc(
        x, aggp[:, :N_NODES, :],
        node_W1[:NODE_DIM], node_W1[NODE_DIM:],
        node_b1.reshape(1, HIDDEN),
        node_W2, node_Wres[:NODE_DIM], node_Wres[NODE_DIM:], nbc,
        node_gamma.reshape(1, NODE_DIM), node_beta.reshape(1, NODE_DIM))
    return (new_x, new_edge)
